# baseline (speedup 1.0000x reference)
import jax
import jax.numpy as jnp
import numpy as np
from jax import lax
from jax.experimental import pallas as pl
from jax.experimental.pallas import tpu as pltpu
from jax.experimental.pallas import tpu_sc as plsc

_POINT_NUM = [2048, 512, 512, 128]
_B = 64
_N1 = 512
_K = 32
_N2 = 128
_EPS = 1e-5

_NC, _NS = 2, 16
_NW = _NC * _NS
_TOT = _B * _N2 * _K
_RPW = _TOT // _NW
_CH = 32
_NBUF = 8
_NPAIR = _RPW // (_CH * _NBUF * 2)

_HI = jax.lax.Precision.HIGHEST


def _k1_body(lc_ref, g_ref, be_ref, b1_ref, w1_ref, h1_ref, stats_ref,
             s_ref, sq_ref):
    i = pl.program_id(0)
    x = lc_ref[0]
    h = jax.lax.dot_general(x, w1_ref[...], (((1,), (0,)), ((), ())),
                            preferred_element_type=jnp.float32,
                            precision=_HI)
    m = h[0:_N1]
    for j in range(1, _K):
        m = jnp.maximum(m, h[j * _N1:(j + 1) * _N1])
    m = m + b1_ref[...]
    h1_ref[0] = jnp.concatenate(
        [m, jnp.zeros((_N1, 64), jnp.float32)], axis=1)

    @pl.when(i == 0)
    def _():
        s_ref[...] = jnp.zeros_like(s_ref)
        sq_ref[...] = jnp.zeros_like(sq_ref)

    s_ref[...] += jnp.sum(m, axis=0, keepdims=True)
    sq_ref[...] += jnp.sum(m * m, axis=0, keepdims=True)

    @pl.when(i == pl.num_programs(0) - 1)
    def _():
        cnt = float(_B * _N1)
        mean = s_ref[...] / cnt
        var = sq_ref[...] / cnt - mean * mean
        scale = g_ref[...] * jax.lax.rsqrt(var + _EPS)
        shift = be_ref[...] - mean * scale
        stats_ref[...] = jnp.concatenate(
            [scale, shift, jnp.zeros((6, 64), jnp.float32)], axis=0)


def _scg_body(nb_ref, h1_ref, gp_ref, idx_all, rows_v, semg, sems):
    c = lax.axis_index("c")
    s = lax.axis_index("s")
    wid = s * _NC + c
    base = wid * _RPW

    pltpu.sync_copy(nb_ref.at[pl.ds(base, _RPW)], idx_all)
    o0 = (base // (_N2 * _K)) * _N1
    half = _RPW // 2

    @pl.loop(0, half // 16)
    def _add0(i):
        sl = pl.ds(i * 16, 16)
        idx_all[sl] = idx_all[sl] + jnp.full((16,), 1, jnp.int32) * o0

    @pl.loop(half // 16, _RPW // 16)
    def _add1(i):
        sl = pl.ds(i * 16, 16)
        idx_all[sl] = idx_all[sl] + jnp.full((16,), 1, jnp.int32) * (o0 + _N1)

    def drain_store(j):
        pltpu.make_async_copy(rows_v.at[j], gp_ref.at[pl.ds(0, _CH)],
                              sems).wait()

    def bank(goff, bufs):
        gathers = []
        for n, j in enumerate(bufs):
            st = goff + n * _CH
            gathers.append(pltpu.async_copy(
                h1_ref.at[idx_all.at[pl.ds(st, _CH)]], rows_v.at[j], semg))
        for gth in gathers:
            gth.wait()
        for n, j in enumerate(bufs):
            st = goff + n * _CH
            pltpu.async_copy(rows_v.at[j], gp_ref.at[pl.ds(base + st, _CH)],
                             sems)

    bufs_a = list(range(_NBUF))
    bufs_b = list(range(_NBUF, 2 * _NBUF))
    span = _NBUF * _CH

    @pl.loop(0, _NPAIR)
    def _pair(g):
        @pl.when(g > 0)
        def _():
            for j in bufs_a:
                drain_store(j)
        bank(g * 2 * span, bufs_a)

        @pl.when(g > 0)
        def _():
            for j in bufs_b:
                drain_store(j)
        bank(g * 2 * span + span, bufs_b)

    for j in bufs_a:
        drain_store(j)
    for j in bufs_b:
        drain_store(j)


def _k3_body(gp_ref, stats_ref, lc3_ref, di1_ref, di3_ref, xyz_ref,
             w2a_ref, w2b_ref, b2_ref, h2_ref, xyz2_ref):
    scale = stats_ref[0:1, :]
    shift = stats_ref[1:2, :]
    gpf = gp_ref[0][:, 0:64]
    gpn = jnp.maximum(gpf * scale + shift, 0.0)
    h2 = (jax.lax.dot_general(gpn.astype(jnp.bfloat16), w2b_ref[...],
                              (((1,), (0,)), ((), ())),
                              preferred_element_type=jnp.float32)
          + jax.lax.dot_general(lc3_ref[0].astype(jnp.bfloat16), w2a_ref[...],
                                (((1,), (0,)), ((), ())),
                                preferred_element_type=jnp.float32))
    m = h2[0:_N2]
    for j in range(1, _K):
        m = jnp.maximum(m, h2[j * _N2:(j + 1) * _N2])
    m = m + b2_ref[...]
    h2_ref[0] = m

    di3c = di3_ref[0]
    iota1 = lax.broadcasted_iota(jnp.int32, (_N2, _N1), 1)
    oh_a = jnp.where(iota1 == di3c, 1.0, 0.0)
    di13 = jax.lax.dot_general(oh_a, di1_ref[0].astype(jnp.float32),
                               (((1,), (0,)), ((), ())),
                               preferred_element_type=jnp.float32,
                               precision=_HI)
    oh_b = jnp.where(iota1.astype(jnp.float32) == di13, 1.0, 0.0)
    xyz2_ref[0] = jax.lax.dot_general(oh_b, xyz_ref[0],
                                      (((1,), (0,)), ((), ())),
                                      preferred_element_type=jnp.float32,
                                      precision=_HI)


def _k4_body(h2_ref, xyz2_ref, g2_ref, be2_ref,
             wm1a_ref, wm1b_ref, bm1_ref, gm1_ref, bem1_ref,
             wm2_ref, bm2_ref, gm2_ref, bem2_ref,
             wf1_ref, bf1_ref, gb1_ref, beb1_ref,
             wf3_ref, bf3_ref, out_ref):

    def bn_rows(x, g, be):
        mean = jnp.mean(x, axis=0, keepdims=True)
        var = jnp.mean(x * x, axis=0, keepdims=True) - mean * mean
        sc = g * jax.lax.rsqrt(var + _EPS)
        return x * sc + (be - mean * sc)

    def mm(a, w):
        return jax.lax.dot_general(a, w, (((1,), (0,)), ((), ())),
                                   preferred_element_type=jnp.float32,
                                   precision=_HI)

    h2n = jnp.maximum(bn_rows(h2_ref[...], g2_ref[...], be2_ref[...]), 0.0)
    m1 = mm(h2n, wm1b_ref[...]) + mm(xyz2_ref[...], wm1a_ref[...]) \
        + bm1_ref[...]
    m1 = jnp.maximum(bn_rows(m1, gm1_ref[...], bem1_ref[...]), 0.0)
    m2 = mm(m1, wm2_ref[...]) + bm2_ref[...]
    m2 = jnp.maximum(bn_rows(m2, gm2_ref[...], bem2_ref[...]), 0.0)

    g = jnp.concatenate(
        [jnp.max(m2[b * _N2:(b + 1) * _N2], axis=0, keepdims=True)
         for b in range(_B)], axis=0)
    x = mm(g, wf1_ref[...]) + bf1_ref[...]
    x = jnp.maximum(bn_rows(x, gb1_ref[...], beb1_ref[...]), 0.0)
    x = mm(x, wf3_ref[...]) + bf3_ref[...]
    mx = jnp.max(x, axis=1, keepdims=True)
    xs = x - mx
    lse = jnp.log(jnp.sum(jnp.exp(xs), axis=1, keepdims=True))
    out_ref[...] = xs - lse


def kernel(xyz, local_coordinates, neighbors, data_idxes,
           W_sa1, b_sa1, g_sa1, be_sa1,
           W_sa2, b_sa2, g_sa2, be_sa2,
           W_m1, b_m1, g_m1, be_m1,
           W_m2, b_m2, g_m2, be_m2,
           W_fc1, b_fc1, g_bn1, be_bn1,
           W_fc3, b_fc3):
    offs = np.cumsum([0] + _POINT_NUM)
    lc1 = local_coordinates[:, offs[1]:offs[2]].transpose(0, 2, 1, 3) \
        .reshape(_B, _N1 * _K, 3)
    lc3 = local_coordinates[:, offs[3]:offs[4]].transpose(0, 2, 1, 3) \
        .reshape(_B, _N2 * _K, 3)
    nb3 = neighbors[:, offs[3]:offs[4]].astype(jnp.int32) \
        .transpose(0, 2, 1).reshape(_TOT)
    di1 = data_idxes[:, offs[1]:offs[2]].astype(jnp.int32).reshape(_B, _N1, 1)
    di3 = data_idxes[:, offs[3]:offs[4]].astype(jnp.int32).reshape(_B, _N2, 1)
    xyz1 = xyz[:, :_N1]

    r1 = lambda v: v.reshape(1, -1)

    h1, stats = pl.pallas_call(
        _k1_body,
        grid=(_B,),
        in_specs=[
            pl.BlockSpec((1, _N1 * _K, 3), lambda i: (i, 0, 0)),
            pl.BlockSpec((1, 64), lambda i: (0, 0)),
            pl.BlockSpec((1, 64), lambda i: (0, 0)),
            pl.BlockSpec((1, 64), lambda i: (0, 0)),
            pl.BlockSpec((3, 64), lambda i: (0, 0)),
        ],
        out_specs=[
            pl.BlockSpec((1, _N1, 128), lambda i: (i, 0, 0)),
            pl.BlockSpec((8, 64), lambda i: (0, 0)),
        ],
        out_shape=[
            jax.ShapeDtypeStruct((_B, _N1, 128), jnp.float32),
            jax.ShapeDtypeStruct((8, 64), jnp.float32),
        ],
        scratch_shapes=[
            pltpu.VMEM((1, 64), jnp.float32),
            pltpu.VMEM((1, 64), jnp.float32),
        ],
    )(lc1, r1(g_sa1), r1(be_sa1), r1(b_sa1), W_sa1)

    h1_flat = h1.reshape(_B * _N1, 128)

    mesh = plsc.VectorSubcoreMesh(core_axis_name="c", subcore_axis_name="s")
    gp = pl.kernel(
        _scg_body,
        out_type=jax.ShapeDtypeStruct((_TOT, 128), jnp.float32),
        mesh=mesh,
        scratch_types=[
            pltpu.VMEM((_RPW,), jnp.int32),
            pltpu.VMEM((2 * _NBUF, _CH, 128), jnp.float32),
            pltpu.SemaphoreType.DMA,
            pltpu.SemaphoreType.DMA,
        ],
    )(nb3, h1_flat)

    gp = gp.reshape(_B, _N2 * _K, 128)

    h2, xyz2 = pl.pallas_call(
        _k3_body,
        grid=(_B,),
        in_specs=[
            pl.BlockSpec((1, _N2 * _K, 128), lambda i: (i, 0, 0)),
            pl.BlockSpec((8, 64), lambda i: (0, 0)),
            pl.BlockSpec((1, _N2 * _K, 3), lambda i: (i, 0, 0)),
            pl.BlockSpec((1, _N1, 1), lambda i: (i, 0, 0)),
            pl.BlockSpec((1, _N2, 1), lambda i: (i, 0, 0)),
            pl.BlockSpec((1, _N1, 3), lambda i: (i, 0, 0)),
            pl.BlockSpec((3, 256), lambda i: (0, 0)),
            pl.BlockSpec((64, 256), lambda i: (0, 0)),
            pl.BlockSpec((1, 256), lambda i: (0, 0)),
        ],
        out_specs=[
            pl.BlockSpec((1, _N2, 256), lambda i: (i, 0, 0)),
            pl.BlockSpec((1, _N2, 3), lambda i: (i, 0, 0)),
        ],
        out_shape=[
            jax.ShapeDtypeStruct((_B, _N2, 256), jnp.float32),
            jax.ShapeDtypeStruct((_B, _N2, 3), jnp.float32),
        ],
    )(gp, stats, lc3, di1, di3, xyz1,
      W_sa2[:3], W_sa2[3:], r1(b_sa2))

    h2r = h2.reshape(_B * _N2, 256)
    xyz2r = xyz2.reshape(_B * _N2, 3)

    full = lambda shape: pl.BlockSpec(shape, lambda: tuple(0 for _ in shape))
    out = pl.pallas_call(
        _k4_body,
        in_specs=[
            full((_B * _N2, 256)), full((_B * _N2, 3)),
            full((1, 256)), full((1, 256)),
            full((3, 256)), full((256, 256)),
            full((1, 256)), full((1, 256)), full((1, 256)),
            full((256, 256)), full((1, 256)), full((1, 256)), full((1, 256)),
            full((256, 128)), full((1, 128)), full((1, 128)), full((1, 128)),
            full((128, 40)), full((1, 40)),
        ],
        out_specs=full((_B, 40)),
        out_shape=jax.ShapeDtypeStruct((_B, 40), jnp.float32),
    )(h2r, xyz2r, r1(g_sa2), r1(be_sa2),
      W_m1[:3], W_m1[3:], r1(b_m1), r1(g_m1), r1(be_m1),
      W_m2, r1(b_m2), r1(g_m2), r1(be_m2),
      W_fc1, r1(b_fc1), r1(g_bn1), r1(be_bn1),
      W_fc3, r1(b_fc3))

    return out


# speedup vs baseline: 1.0002x; 1.0002x over previous
import jax
import jax.numpy as jnp
import numpy as np
from jax import lax
from jax.experimental import pallas as pl
from jax.experimental.pallas import tpu as pltpu
from jax.experimental.pallas import tpu_sc as plsc

_POINT_NUM = [2048, 512, 512, 128]
_B = 64
_N1 = 512
_K = 32
_N2 = 128
_EPS = 1e-5

_NC, _NS = 2, 16
_NW = _NC * _NS
_TOT = _B * _N2 * _K
_RPW = _TOT // _NW
_CH = 32
_NBUF = 8
_NPAIR = _RPW // (_CH * _NBUF * 2)

_HI = jax.lax.Precision.HIGHEST


def _k1_body(lc_ref, g_ref, be_ref, b1_ref, w1_ref, h1_ref, stats_ref,
             s_ref, sq_ref):
    i = pl.program_id(0)
    x = lc_ref[0]
    h = jax.lax.dot_general(x, w1_ref[...], (((1,), (0,)), ((), ())),
                            preferred_element_type=jnp.float32,
                            precision=_HI)
    m = h[0:_N1]
    for j in range(1, _K):
        m = jnp.maximum(m, h[j * _N1:(j + 1) * _N1])
    m = m + b1_ref[...]
    h1_ref[...] = jnp.concatenate(
        [m, jnp.zeros((_N1, 64), jnp.float32)], axis=1)

    @pl.when(i == 0)
    def _():
        s_ref[...] = jnp.zeros_like(s_ref)
        sq_ref[...] = jnp.zeros_like(sq_ref)

    s_ref[...] += jnp.sum(m, axis=0, keepdims=True)
    sq_ref[...] += jnp.sum(m * m, axis=0, keepdims=True)

    @pl.when(i == pl.num_programs(0) - 1)
    def _():
        cnt = float(_B * _N1)
        mean = s_ref[...] / cnt
        var = sq_ref[...] / cnt - mean * mean
        scale = g_ref[...] * jax.lax.rsqrt(var + _EPS)
        shift = be_ref[...] - mean * scale
        stats_ref[...] = jnp.concatenate(
            [scale, shift, jnp.zeros((6, 64), jnp.float32)], axis=0)


def _scg_body(nb_ref, h1_ref, gp_ref, idx_all, rows_v, semg, sems):
    c = lax.axis_index("c")
    s = lax.axis_index("s")
    wid = s * _NC + c
    base = wid * _RPW

    pltpu.sync_copy(nb_ref.at[pl.ds(base, _RPW)], idx_all)
    o0 = (base // (_N2 * _K)) * _N1
    half = _RPW // 2

    @pl.loop(0, half // 16)
    def _add0(i):
        sl = pl.ds(i * 16, 16)
        idx_all[sl] = idx_all[sl] + jnp.full((16,), 1, jnp.int32) * o0

    @pl.loop(half // 16, _RPW // 16)
    def _add1(i):
        sl = pl.ds(i * 16, 16)
        idx_all[sl] = idx_all[sl] + jnp.full((16,), 1, jnp.int32) * (o0 + _N1)

    def drain_store(j):
        pltpu.make_async_copy(rows_v.at[j], gp_ref.at[pl.ds(0, _CH)],
                              sems).wait()

    def bank(goff, bufs):
        gathers = []
        for n, j in enumerate(bufs):
            st = goff + n * _CH
            gathers.append(pltpu.async_copy(
                h1_ref.at[idx_all.at[pl.ds(st, _CH)]], rows_v.at[j], semg))
        for gth in gathers:
            gth.wait()
        for n, j in enumerate(bufs):
            st = goff + n * _CH
            pltpu.async_copy(rows_v.at[j], gp_ref.at[pl.ds(base + st, _CH)],
                             sems)

    bufs_a = list(range(_NBUF))
    bufs_b = list(range(_NBUF, 2 * _NBUF))
    span = _NBUF * _CH

    @pl.loop(0, _NPAIR)
    def _pair(g):
        @pl.when(g > 0)
        def _():
            for j in bufs_a:
                drain_store(j)
        bank(g * 2 * span, bufs_a)

        @pl.when(g > 0)
        def _():
            for j in bufs_b:
                drain_store(j)
        bank(g * 2 * span + span, bufs_b)

    for j in bufs_a:
        drain_store(j)
    for j in bufs_b:
        drain_store(j)


def _k3_body(gp_ref, stats_ref, lc3_ref, di1_ref, di3_ref, xyz_ref,
             w2a_ref, w2b_ref, b2_ref, h2_ref, xyz2_ref):
    scale = stats_ref[0:1, :]
    shift = stats_ref[1:2, :]
    gpf = gp_ref[:, 0:64]
    gpn = jnp.maximum(gpf * scale + shift, 0.0)
    h2 = (jax.lax.dot_general(gpn.astype(jnp.bfloat16), w2b_ref[...],
                              (((1,), (0,)), ((), ())),
                              preferred_element_type=jnp.float32)
          + jax.lax.dot_general(lc3_ref[0].astype(jnp.bfloat16), w2a_ref[...],
                                (((1,), (0,)), ((), ())),
                                preferred_element_type=jnp.float32))
    m = h2[0:_N2]
    for j in range(1, _K):
        m = jnp.maximum(m, h2[j * _N2:(j + 1) * _N2])
    m = m + b2_ref[...]
    h2_ref[...] = m

    di3c = di3_ref[0]
    iota1 = lax.broadcasted_iota(jnp.int32, (_N2, _N1), 1)
    oh_a = jnp.where(iota1 == di3c, 1.0, 0.0)
    di13 = jax.lax.dot_general(oh_a, di1_ref[0].astype(jnp.float32),
                               (((1,), (0,)), ((), ())),
                               preferred_element_type=jnp.float32,
                               precision=_HI)
    oh_b = jnp.where(iota1.astype(jnp.float32) == di13, 1.0, 0.0)
    xyz2_ref[...] = jax.lax.dot_general(oh_b, xyz_ref[0],
                                      (((1,), (0,)), ((), ())),
                                      preferred_element_type=jnp.float32,
                                      precision=_HI)


def _k4_body(h2_ref, xyz2_ref, g2_ref, be2_ref,
             wm1a_ref, wm1b_ref, bm1_ref, gm1_ref, bem1_ref,
             wm2_ref, bm2_ref, gm2_ref, bem2_ref,
             wf1_ref, bf1_ref, gb1_ref, beb1_ref,
             wf3_ref, bf3_ref, out_ref):

    def bn_rows(x, g, be):
        mean = jnp.mean(x, axis=0, keepdims=True)
        var = jnp.mean(x * x, axis=0, keepdims=True) - mean * mean
        sc = g * jax.lax.rsqrt(var + _EPS)
        return x * sc + (be - mean * sc)

    def mm(a, w):
        return jax.lax.dot_general(a, w, (((1,), (0,)), ((), ())),
                                   preferred_element_type=jnp.float32,
                                   precision=_HI)

    h2n = jnp.maximum(bn_rows(h2_ref[...], g2_ref[...], be2_ref[...]), 0.0)
    m1 = mm(h2n, wm1b_ref[...]) + mm(xyz2_ref[...], wm1a_ref[...]) \
        + bm1_ref[...]
    m1 = jnp.maximum(bn_rows(m1, gm1_ref[...], bem1_ref[...]), 0.0)
    m2 = mm(m1, wm2_ref[...]) + bm2_ref[...]
    m2 = jnp.maximum(bn_rows(m2, gm2_ref[...], bem2_ref[...]), 0.0)

    g = jnp.concatenate(
        [jnp.max(m2[b * _N2:(b + 1) * _N2], axis=0, keepdims=True)
         for b in range(_B)], axis=0)
    x = mm(g, wf1_ref[...]) + bf1_ref[...]
    x = jnp.maximum(bn_rows(x, gb1_ref[...], beb1_ref[...]), 0.0)
    x = mm(x, wf3_ref[...]) + bf3_ref[...]
    mx = jnp.max(x, axis=1, keepdims=True)
    xs = x - mx
    lse = jnp.log(jnp.sum(jnp.exp(xs), axis=1, keepdims=True))
    out_ref[...] = xs - lse


def kernel(xyz, local_coordinates, neighbors, data_idxes,
           W_sa1, b_sa1, g_sa1, be_sa1,
           W_sa2, b_sa2, g_sa2, be_sa2,
           W_m1, b_m1, g_m1, be_m1,
           W_m2, b_m2, g_m2, be_m2,
           W_fc1, b_fc1, g_bn1, be_bn1,
           W_fc3, b_fc3):
    offs = np.cumsum([0] + _POINT_NUM)
    lc1 = local_coordinates[:, offs[1]:offs[2]].transpose(0, 2, 1, 3) \
        .reshape(_B, _N1 * _K, 3)
    lc3 = local_coordinates[:, offs[3]:offs[4]].transpose(0, 2, 1, 3) \
        .reshape(_B, _N2 * _K, 3)
    nb3 = neighbors[:, offs[3]:offs[4]].astype(jnp.int32) \
        .transpose(0, 2, 1).reshape(_TOT)
    di1 = data_idxes[:, offs[1]:offs[2]].astype(jnp.int32).reshape(_B, _N1, 1)
    di3 = data_idxes[:, offs[3]:offs[4]].astype(jnp.int32).reshape(_B, _N2, 1)
    xyz1 = xyz[:, :_N1]

    r1 = lambda v: v.reshape(1, -1)

    h1, stats = pl.pallas_call(
        _k1_body,
        grid=(_B,),
        in_specs=[
            pl.BlockSpec((1, _N1 * _K, 3), lambda i: (i, 0, 0)),
            pl.BlockSpec((1, 64), lambda i: (0, 0)),
            pl.BlockSpec((1, 64), lambda i: (0, 0)),
            pl.BlockSpec((1, 64), lambda i: (0, 0)),
            pl.BlockSpec((3, 64), lambda i: (0, 0)),
        ],
        out_specs=[
            pl.BlockSpec((_N1, 128), lambda i: (i, 0)),
            pl.BlockSpec((8, 64), lambda i: (0, 0)),
        ],
        out_shape=[
            jax.ShapeDtypeStruct((_B * _N1, 128), jnp.float32),
            jax.ShapeDtypeStruct((8, 64), jnp.float32),
        ],
        scratch_shapes=[
            pltpu.VMEM((1, 64), jnp.float32),
            pltpu.VMEM((1, 64), jnp.float32),
        ],
    )(lc1, r1(g_sa1), r1(be_sa1), r1(b_sa1), W_sa1)


    mesh = plsc.VectorSubcoreMesh(core_axis_name="c", subcore_axis_name="s")
    gp = pl.kernel(
        _scg_body,
        out_type=jax.ShapeDtypeStruct((_TOT, 128), jnp.float32),
        mesh=mesh,
        scratch_types=[
            pltpu.VMEM((_RPW,), jnp.int32),
            pltpu.VMEM((2 * _NBUF, _CH, 128), jnp.float32),
            pltpu.SemaphoreType.DMA,
            pltpu.SemaphoreType.DMA,
        ],
    )(nb3, h1)


    h2, xyz2 = pl.pallas_call(
        _k3_body,
        grid=(_B,),
        in_specs=[
            pl.BlockSpec((_N2 * _K, 128), lambda i: (i, 0)),
            pl.BlockSpec((8, 64), lambda i: (0, 0)),
            pl.BlockSpec((1, _N2 * _K, 3), lambda i: (i, 0, 0)),
            pl.BlockSpec((1, _N1, 1), lambda i: (i, 0, 0)),
            pl.BlockSpec((1, _N2, 1), lambda i: (i, 0, 0)),
            pl.BlockSpec((1, _N1, 3), lambda i: (i, 0, 0)),
            pl.BlockSpec((3, 256), lambda i: (0, 0)),
            pl.BlockSpec((64, 256), lambda i: (0, 0)),
            pl.BlockSpec((1, 256), lambda i: (0, 0)),
        ],
        out_specs=[
            pl.BlockSpec((_N2, 256), lambda i: (i, 0)),
            pl.BlockSpec((_N2, 3), lambda i: (i, 0)),
        ],
        out_shape=[
            jax.ShapeDtypeStruct((_B * _N2, 256), jnp.float32),
            jax.ShapeDtypeStruct((_B * _N2, 3), jnp.float32),
        ],
    )(gp, stats, lc3, di1, di3, xyz1,
      W_sa2[:3], W_sa2[3:], r1(b_sa2))


    full = lambda shape: pl.BlockSpec(shape, lambda: tuple(0 for _ in shape))
    out = pl.pallas_call(
        _k4_body,
        in_specs=[
            full((_B * _N2, 256)), full((_B * _N2, 3)),
            full((1, 256)), full((1, 256)),
            full((3, 256)), full((256, 256)),
            full((1, 256)), full((1, 256)), full((1, 256)),
            full((256, 256)), full((1, 256)), full((1, 256)), full((1, 256)),
            full((256, 128)), full((1, 128)), full((1, 128)), full((1, 128)),
            full((128, 40)), full((1, 40)),
        ],
        out_specs=full((_B, 40)),
        out_shape=jax.ShapeDtypeStruct((_B, 40), jnp.float32),
    )(h2, xyz2, r1(g_sa2), r1(be_sa2),
      W_m1[:3], W_m1[3:], r1(b_m1), r1(g_m1), r1(be_m1),
      W_m2, r1(b_m2), r1(g_m2), r1(be_m2),
      W_fc1, r1(b_fc1), r1(g_bn1), r1(be_bn1),
      W_fc3, r1(b_fc3))

    return out


# speedup vs baseline: 1.6556x; 1.6552x over previous
import jax
import jax.numpy as jnp
import numpy as np
from jax import lax
from jax.experimental import pallas as pl
from jax.experimental.pallas import tpu as pltpu
from jax.experimental.pallas import tpu_sc as plsc

_POINT_NUM = [2048, 512, 512, 128]
_B = 64
_N1 = 512
_K = 32
_N2 = 128
_EPS = 1e-5

_NC, _NS = 2, 16
_NW = _NC * _NS
_TOT = _B * _N2 * _K
_RPW = _TOT // _NW
_CH = 32
_NBUF = 8
_NPAIR = _RPW // (_CH * _NBUF * 2)

_HI = jax.lax.Precision.HIGHEST


def _k1_body(lc_ref, g_ref, be_ref, b1_ref, w1_ref, h1_ref, stats_ref,
             s_ref, sq_ref):
    i = pl.program_id(0)
    x = lc_ref[0].reshape(_N1 * _K, 3)
    h = jax.lax.dot_general(x, w1_ref[...], (((1,), (0,)), ((), ())),
                            preferred_element_type=jnp.float32,
                            precision=_HI)
    m = jnp.max(h.reshape(_N1, _K, 64), axis=1) + b1_ref[...]
    h1_ref[...] = jnp.concatenate(
        [m, jnp.zeros((_N1, 64), jnp.float32)], axis=1)

    @pl.when(i == 0)
    def _():
        s_ref[...] = jnp.zeros_like(s_ref)
        sq_ref[...] = jnp.zeros_like(sq_ref)

    s_ref[...] += jnp.sum(m, axis=0, keepdims=True)
    sq_ref[...] += jnp.sum(m * m, axis=0, keepdims=True)

    @pl.when(i == pl.num_programs(0) - 1)
    def _():
        cnt = float(_B * _N1)
        mean = s_ref[...] / cnt
        var = sq_ref[...] / cnt - mean * mean
        scale = g_ref[...] * jax.lax.rsqrt(var + _EPS)
        shift = be_ref[...] - mean * scale
        stats_ref[...] = jnp.concatenate(
            [scale, shift, jnp.zeros((6, 64), jnp.float32)], axis=0)


def _scg_body(nb_ref, h1_ref, gp_ref, idx_all, rows_v, semg, sems):
    c = lax.axis_index("c")
    s = lax.axis_index("s")
    wid = s * _NC + c
    base = wid * _RPW

    pltpu.sync_copy(nb_ref.at[pl.ds(base, _RPW)], idx_all)
    o0 = (base // (_N2 * _K)) * _N1
    half = _RPW // 2

    @pl.loop(0, half // 16)
    def _add0(i):
        sl = pl.ds(i * 16, 16)
        idx_all[sl] = idx_all[sl] + jnp.full((16,), 1, jnp.int32) * o0

    @pl.loop(half // 16, _RPW // 16)
    def _add1(i):
        sl = pl.ds(i * 16, 16)
        idx_all[sl] = idx_all[sl] + jnp.full((16,), 1, jnp.int32) * (o0 + _N1)

    def drain_store(j):
        pltpu.make_async_copy(rows_v.at[j], gp_ref.at[pl.ds(0, _CH)],
                              sems).wait()

    def bank(goff, bufs):
        gathers = []
        for n, j in enumerate(bufs):
            st = goff + n * _CH
            gathers.append(pltpu.async_copy(
                h1_ref.at[idx_all.at[pl.ds(st, _CH)]], rows_v.at[j], semg))
        for gth in gathers:
            gth.wait()
        for n, j in enumerate(bufs):
            st = goff + n * _CH
            pltpu.async_copy(rows_v.at[j], gp_ref.at[pl.ds(base + st, _CH)],
                             sems)

    bufs_a = list(range(_NBUF))
    bufs_b = list(range(_NBUF, 2 * _NBUF))
    span = _NBUF * _CH

    @pl.loop(0, _NPAIR)
    def _pair(g):
        @pl.when(g > 0)
        def _():
            for j in bufs_a:
                drain_store(j)
        bank(g * 2 * span, bufs_a)

        @pl.when(g > 0)
        def _():
            for j in bufs_b:
                drain_store(j)
        bank(g * 2 * span + span, bufs_b)

    for j in bufs_a:
        drain_store(j)
    for j in bufs_b:
        drain_store(j)


def _k3_body(gp_ref, stats_ref, lc3_ref, di1_ref, di3_ref, xyz_ref,
             w2a_ref, w2b_ref, b2_ref, h2_ref, xyz2_ref):
    scale = stats_ref[0:1, :]
    shift = stats_ref[1:2, :]
    gpf = gp_ref[:, 0:64]
    gpn = jnp.maximum(gpf * scale + shift, 0.0)
    h2 = (jax.lax.dot_general(gpn.astype(jnp.bfloat16), w2b_ref[...],
                              (((1,), (0,)), ((), ())),
                              preferred_element_type=jnp.float32)
          + jax.lax.dot_general(lc3_ref[0].reshape(_N2 * _K, 3)
                                .astype(jnp.bfloat16), w2a_ref[...],
                                (((1,), (0,)), ((), ())),
                                preferred_element_type=jnp.float32))
    m = jnp.max(h2.reshape(_N2, _K, 256), axis=1) + b2_ref[...]
    h2_ref[...] = m

    di3c = di3_ref[0]
    iota1 = lax.broadcasted_iota(jnp.int32, (_N2, _N1), 1)
    oh_a = jnp.where(iota1 == di3c, 1.0, 0.0)
    di13 = jax.lax.dot_general(oh_a, di1_ref[0].astype(jnp.float32),
                               (((1,), (0,)), ((), ())),
                               preferred_element_type=jnp.float32,
                               precision=_HI)
    oh_b = jnp.where(iota1.astype(jnp.float32) == di13, 1.0, 0.0)
    xyz2_ref[...] = jax.lax.dot_general(oh_b, xyz_ref[0],
                                      (((1,), (0,)), ((), ())),
                                      preferred_element_type=jnp.float32,
                                      precision=_HI)


def _k4_body(h2_ref, xyz2_ref, g2_ref, be2_ref,
             wm1a_ref, wm1b_ref, bm1_ref, gm1_ref, bem1_ref,
             wm2_ref, bm2_ref, gm2_ref, bem2_ref,
             wf1_ref, bf1_ref, gb1_ref, beb1_ref,
             wf3_ref, bf3_ref, out_ref):

    def bn_rows(x, g, be):
        mean = jnp.mean(x, axis=0, keepdims=True)
        var = jnp.mean(x * x, axis=0, keepdims=True) - mean * mean
        sc = g * jax.lax.rsqrt(var + _EPS)
        return x * sc + (be - mean * sc)

    def mm(a, w):
        return jax.lax.dot_general(a, w, (((1,), (0,)), ((), ())),
                                   preferred_element_type=jnp.float32,
                                   precision=_HI)

    h2n = jnp.maximum(bn_rows(h2_ref[...], g2_ref[...], be2_ref[...]), 0.0)
    m1 = mm(h2n, wm1b_ref[...]) + mm(xyz2_ref[...], wm1a_ref[...]) \
        + bm1_ref[...]
    m1 = jnp.maximum(bn_rows(m1, gm1_ref[...], bem1_ref[...]), 0.0)
    m2 = mm(m1, wm2_ref[...]) + bm2_ref[...]
    m2 = jnp.maximum(bn_rows(m2, gm2_ref[...], bem2_ref[...]), 0.0)

    g = jnp.concatenate(
        [jnp.max(m2[b * _N2:(b + 1) * _N2], axis=0, keepdims=True)
         for b in range(_B)], axis=0)
    x = mm(g, wf1_ref[...]) + bf1_ref[...]
    x = jnp.maximum(bn_rows(x, gb1_ref[...], beb1_ref[...]), 0.0)
    x = mm(x, wf3_ref[...]) + bf3_ref[...]
    mx = jnp.max(x, axis=1, keepdims=True)
    xs = x - mx
    lse = jnp.log(jnp.sum(jnp.exp(xs), axis=1, keepdims=True))
    out_ref[...] = xs - lse


def kernel(xyz, local_coordinates, neighbors, data_idxes,
           W_sa1, b_sa1, g_sa1, be_sa1,
           W_sa2, b_sa2, g_sa2, be_sa2,
           W_m1, b_m1, g_m1, be_m1,
           W_m2, b_m2, g_m2, be_m2,
           W_fc1, b_fc1, g_bn1, be_bn1,
           W_fc3, b_fc3):
    offs = np.cumsum([0] + _POINT_NUM)
    lc1 = local_coordinates[:, offs[1]:offs[2]]
    lc3 = local_coordinates[:, offs[3]:offs[4]]
    nb3 = neighbors[:, offs[3]:offs[4]].astype(jnp.int32).reshape(_TOT)
    di1 = data_idxes[:, offs[1]:offs[2]].astype(jnp.int32).reshape(_B, _N1, 1)
    di3 = data_idxes[:, offs[3]:offs[4]].astype(jnp.int32).reshape(_B, _N2, 1)
    xyz1 = xyz[:, :_N1]

    r1 = lambda v: v.reshape(1, -1)

    h1, stats = pl.pallas_call(
        _k1_body,
        grid=(_B,),
        in_specs=[
            pl.BlockSpec((1, _N1, _K, 3), lambda i: (i, 0, 0, 0)),
            pl.BlockSpec((1, 64), lambda i: (0, 0)),
            pl.BlockSpec((1, 64), lambda i: (0, 0)),
            pl.BlockSpec((1, 64), lambda i: (0, 0)),
            pl.BlockSpec((3, 64), lambda i: (0, 0)),
        ],
        out_specs=[
            pl.BlockSpec((_N1, 128), lambda i: (i, 0)),
            pl.BlockSpec((8, 64), lambda i: (0, 0)),
        ],
        out_shape=[
            jax.ShapeDtypeStruct((_B * _N1, 128), jnp.float32),
            jax.ShapeDtypeStruct((8, 64), jnp.float32),
        ],
        scratch_shapes=[
            pltpu.VMEM((1, 64), jnp.float32),
            pltpu.VMEM((1, 64), jnp.float32),
        ],
    )(lc1, r1(g_sa1), r1(be_sa1), r1(b_sa1), W_sa1)


    mesh = plsc.VectorSubcoreMesh(core_axis_name="c", subcore_axis_name="s")
    gp = pl.kernel(
        _scg_body,
        out_type=jax.ShapeDtypeStruct((_TOT, 128), jnp.float32),
        mesh=mesh,
        scratch_types=[
            pltpu.VMEM((_RPW,), jnp.int32),
            pltpu.VMEM((2 * _NBUF, _CH, 128), jnp.float32),
            pltpu.SemaphoreType.DMA,
            pltpu.SemaphoreType.DMA,
        ],
    )(nb3, h1)


    h2, xyz2 = pl.pallas_call(
        _k3_body,
        grid=(_B,),
        in_specs=[
            pl.BlockSpec((_N2 * _K, 128), lambda i: (i, 0)),
            pl.BlockSpec((8, 64), lambda i: (0, 0)),
            pl.BlockSpec((1, _N2, _K, 3), lambda i: (i, 0, 0, 0)),
            pl.BlockSpec((1, _N1, 1), lambda i: (i, 0, 0)),
            pl.BlockSpec((1, _N2, 1), lambda i: (i, 0, 0)),
            pl.BlockSpec((1, _N1, 3), lambda i: (i, 0, 0)),
            pl.BlockSpec((3, 256), lambda i: (0, 0)),
            pl.BlockSpec((64, 256), lambda i: (0, 0)),
            pl.BlockSpec((1, 256), lambda i: (0, 0)),
        ],
        out_specs=[
            pl.BlockSpec((_N2, 256), lambda i: (i, 0)),
            pl.BlockSpec((_N2, 3), lambda i: (i, 0)),
        ],
        out_shape=[
            jax.ShapeDtypeStruct((_B * _N2, 256), jnp.float32),
            jax.ShapeDtypeStruct((_B * _N2, 3), jnp.float32),
        ],
    )(gp, stats, lc3, di1, di3, xyz1,
      W_sa2[:3], W_sa2[3:], r1(b_sa2))


    full = lambda shape: pl.BlockSpec(shape, lambda: tuple(0 for _ in shape))
    out = pl.pallas_call(
        _k4_body,
        in_specs=[
            full((_B * _N2, 256)), full((_B * _N2, 3)),
            full((1, 256)), full((1, 256)),
            full((3, 256)), full((256, 256)),
            full((1, 256)), full((1, 256)), full((1, 256)),
            full((256, 256)), full((1, 256)), full((1, 256)), full((1, 256)),
            full((256, 128)), full((1, 128)), full((1, 128)), full((1, 128)),
            full((128, 40)), full((1, 40)),
        ],
        out_specs=full((_B, 40)),
        out_shape=jax.ShapeDtypeStruct((_B, 40), jnp.float32),
    )(h2, xyz2, r1(g_sa2), r1(be_sa2),
      W_m1[:3], W_m1[3:], r1(b_m1), r1(g_m1), r1(be_m1),
      W_m2, r1(b_m2), r1(g_m2), r1(be_m2),
      W_fc1, r1(b_fc1), r1(g_bn1), r1(be_bn1),
      W_fc3, r1(b_fc3))

    return out
